# Initial kernel scaffold; baseline (speedup 1.0000x reference)
import functools

import jax
import jax.numpy as jnp
from jax import lax
from jax.experimental import pallas as pl
from jax.experimental.pallas import tpu as pltpu
from jax.experimental.pallas import tpu_sc as plsc

_EPS = 1e-5
_TE = 800
_GW = 128
_CH = 128


def _leaky(h):
    return jnp.where(h >= 0, h, 0.01 * h)


def _sc_gather(x, src):
    n, nd = x.shape
    e = src.shape[0]
    assert e % _GW == 0
    idx = src.reshape(1, e)
    mesh = plsc.VectorSubcoreMesh(core_axis_name="core", subcore_axis_name="subcore")

    @functools.partial(
        pl.kernel,
        out_type=jax.ShapeDtypeStruct((e, nd), jnp.float32),
        mesh=mesh,
    )
    def k(x_hbm, i_hbm, o_hbm):
        def body(i_vmem, o_vmem):
            pltpu.sync_copy(x_hbm.at[i_vmem.at[0]], o_vmem)

        pltpu.emit_pipeline(
            body,
            grid=(e // _GW,),
            in_specs=[pl.BlockSpec((1, _GW), index_map=lambda i: (0, i))],
            out_specs=[pl.BlockSpec((_GW, nd), index_map=lambda i: (i, 0))],
            core_axis_name=("core", "subcore"),
            dimension_semantics=(pltpu.PARALLEL,),
        )(i_hbm, o_hbm)

    return k(x, idx)


def _sc_scatter_add(m, dst, n):
    e, od = m.shape
    e_half = e // 2
    epc = e_half // 16
    n_full = epc // _CH
    tail = epc - n_full * _CH
    nz = n // 16
    assert epc % 8 == 0 and n % 16 == 0 and tail % 8 == 0
    mesh = plsc.VectorSubcoreMesh(core_axis_name="core", subcore_axis_name="subcore")

    scratch = [
        pltpu.VMEM((_CH,), jnp.int32),
        pltpu.VMEM((_CH, od), jnp.float32),
        pltpu.VMEM((nz, od), jnp.float32),
        pltpu.VMEM_SHARED((n, od), jnp.float32),
    ]
    if tail:
        scratch += [
            pltpu.VMEM((tail,), jnp.int32),
            pltpu.VMEM((tail, od), jnp.float32),
        ]

    @functools.partial(
        pl.kernel,
        out_type=jax.ShapeDtypeStruct((2, n, od), jnp.float32),
        mesh=mesh,
        scratch_types=scratch,
    )
    def k(m_hbm, d_hbm, o_hbm, idx_v, rows_v, z_v, acc_sh, *tail_refs):
        c = lax.axis_index("core")
        s = lax.axis_index("subcore")
        base = c * e_half + s * epc

        @pl.loop(0, nz)
        def _(i):
            z_v[i, :] = jnp.zeros((od,), jnp.float32)

        pltpu.sync_copy(z_v, acc_sh.at[pl.ds(s * nz, nz)])
        plsc.subcore_barrier()

        @pl.loop(0, n_full)
        def _(j):
            off = base + j * _CH
            pltpu.sync_copy(d_hbm.at[pl.ds(off, _CH)], idx_v)
            pltpu.sync_copy(m_hbm.at[pl.ds(off, _CH)], rows_v)
            pltpu.sync_copy(rows_v, acc_sh.at[idx_v], add=True)

        if tail:
            idxt_v, rowst_v = tail_refs
            offt = base + n_full * _CH
            pltpu.sync_copy(d_hbm.at[pl.ds(offt, tail)], idxt_v)
            pltpu.sync_copy(m_hbm.at[pl.ds(offt, tail)], rowst_v)
            pltpu.sync_copy(rowst_v, acc_sh.at[idxt_v], add=True)

        plsc.subcore_barrier()
        pltpu.sync_copy(acc_sh.at[pl.ds(s * nz, nz)], o_hbm.at[c, pl.ds(s * nz, nz)])

    return k(m, dst)


def _mlp_body(ea_ref, xj_ref, w1_ref, b1_ref, w2_ref, b2_ref, m_ref):
    h1 = jnp.dot(ea_ref[...], w1_ref[...], preferred_element_type=jnp.float32)
    h1 = _leaky(h1 + b1_ref[...])
    h2 = jnp.dot(h1, w2_ref[...], preferred_element_type=jnp.float32)
    h2 = _leaky(h2 + b2_ref[...])
    te, nd = xj_ref.shape
    od = m_ref.shape[-1]
    prod = h2.reshape(te, od, nd) * xj_ref[...].reshape(te, 1, nd)
    m_ref[...] = jnp.sum(prod, axis=-1)


def _tc_mlp(edge_attr, x_j, w1f, b1f, w2p, b2p, od):
    e, ed = edge_attr.shape
    nd = x_j.shape[1]
    hd = w1f.shape[1]
    assert e % _TE == 0
    return pl.pallas_call(
        _mlp_body,
        grid=(e // _TE,),
        in_specs=[
            pl.BlockSpec((_TE, ed), lambda i: (i, 0)),
            pl.BlockSpec((_TE, nd), lambda i: (i, 0)),
            pl.BlockSpec((ed, hd), lambda i: (0, 0)),
            pl.BlockSpec((1, hd), lambda i: (0, 0)),
            pl.BlockSpec((nd, hd * od), lambda i: (0, 0)),
            pl.BlockSpec((1, hd * od), lambda i: (0, 0)),
        ],
        out_specs=pl.BlockSpec((_TE, od), lambda i: (i, 0)),
        out_shape=jax.ShapeDtypeStruct((e, od), jnp.float32),
    )(edge_attr, x_j, w1f, b1f, w2p, b2p)


def _finish_body(p_ref, x_ref, rw_ref, b_ref, o_ref):
    root = jnp.dot(x_ref[...], rw_ref[...], preferred_element_type=jnp.float32)
    o_ref[...] = p_ref[0] + p_ref[1] + root + b_ref[...]


def _tc_finish(partials, x, root_w, bias):
    n, nd = x.shape
    od = root_w.shape[1]
    return pl.pallas_call(
        _finish_body,
        out_shape=jax.ShapeDtypeStruct((n, od), jnp.float32),
    )(partials, x, root_w, bias.reshape(1, od))


def kernel(x, edge_index, edge_attr, batch, lin1_w, lin1_b, bn1_g, bn1_b,
           bn1_rm, bn1_rv, lin2_w, lin2_b, bn2_g, bn2_b, bn2_rm, bn2_rv,
           root_w, bias):
    n, nd = x.shape
    e, ed = edge_attr.shape
    hd = lin1_w.shape[1]
    od = root_w.shape[1]
    src = edge_index[0]
    dst = edge_index[1]

    s1 = bn1_g * lax.rsqrt(bn1_rv + _EPS)
    w1f = lin1_w * s1
    b1f = (s1 * (lin1_b - bn1_rm) + bn1_b).reshape(1, hd)
    s2 = bn2_g * lax.rsqrt(bn2_rv + _EPS)
    w2f = lin2_w * s2
    b2f = s2 * (lin2_b - bn2_rm) + bn2_b
    w2p = w2f.reshape(hd, hd, od).transpose(0, 2, 1).reshape(hd, hd * od)
    b2p = b2f.reshape(hd, od).T.reshape(1, hd * od)

    x_j = _sc_gather(x, src)
    m = _tc_mlp(edge_attr, x_j, w1f, b1f, w2p, b2p, od)
    partials = _sc_scatter_add(m, dst, n)
    return _tc_finish(partials, x, root_w, bias)


# speedup vs baseline: 3.2172x; 3.2172x over previous
import functools

import jax
import jax.numpy as jnp
from jax import lax
from jax.experimental import pallas as pl
from jax.experimental.pallas import tpu as pltpu
from jax.experimental.pallas import tpu_sc as plsc

_EPS = 1e-5
_TE = 1280
_GW = 128
_CH = 128
_SPLIT = 81920


def _leaky(h):
    return jnp.maximum(h, 0.01 * h)


def _sc_gather(x, idx2, e0, ec):
    n, nd = x.shape
    assert ec % _GW == 0 and e0 % _GW == 0
    w0 = e0 // _GW
    mesh = plsc.VectorSubcoreMesh(core_axis_name="core", subcore_axis_name="subcore")

    @functools.partial(
        pl.kernel,
        out_type=jax.ShapeDtypeStruct((ec, nd), jnp.float32),
        mesh=mesh,
    )
    def k(x_hbm, i_hbm, o_hbm):
        def body(i_vmem, o_vmem):
            pltpu.sync_copy(x_hbm.at[i_vmem.at[0]], o_vmem)

        pltpu.emit_pipeline(
            body,
            grid=(ec // _GW,),
            in_specs=[pl.BlockSpec((1, _GW), index_map=lambda i: (0, i + w0))],
            out_specs=[pl.BlockSpec((_GW, nd), index_map=lambda i: (i, 0))],
            core_axis_name=("core", "subcore"),
            dimension_semantics=(pltpu.PARALLEL,),
        )(i_hbm, o_hbm)

    return k(x, idx2)


def _sc_scatter_add(m, dst, n, e0):
    ec, od = m.shape
    e_half = ec // 2
    epc = e_half // 16
    n_full = epc // _CH
    tail = epc - n_full * _CH
    nz = -(-n // (16 * 8)) * 8
    npad = nz * 16
    assert epc % 8 == 0 and tail % 8 == 0 and e0 % 8 == 0
    mesh = plsc.VectorSubcoreMesh(core_axis_name="core", subcore_axis_name="subcore")

    scratch = [
        pltpu.VMEM((_CH,), jnp.int32),
        pltpu.VMEM((_CH, od), jnp.float32),
        pltpu.VMEM((nz, od), jnp.float32),
        pltpu.VMEM_SHARED((npad, od), jnp.float32),
    ]
    if tail:
        scratch += [
            pltpu.VMEM((tail,), jnp.int32),
            pltpu.VMEM((tail, od), jnp.float32),
        ]

    @functools.partial(
        pl.kernel,
        out_type=jax.ShapeDtypeStruct((2, npad, od), jnp.float32),
        mesh=mesh,
        scratch_types=scratch,
        compiler_params=pltpu.CompilerParams(use_tc_tiling_on_sc=False),
    )
    def k(m_hbm, d_hbm, o_hbm, idx_v, rows_v, z_v, acc_sh, *tail_refs):
        c = lax.axis_index("core")
        s = lax.axis_index("subcore")
        mbase = c * e_half + s * epc
        dbase = e0 + mbase

        @pl.loop(0, nz)
        def _(i):
            z_v[i, :] = jnp.zeros((od,), jnp.float32)

        pltpu.sync_copy(z_v, acc_sh.at[pl.ds(s * nz, nz)])
        plsc.subcore_barrier()

        @pl.loop(0, n_full)
        def _(j):
            pltpu.sync_copy(d_hbm.at[pl.ds(dbase + j * _CH, _CH)], idx_v)
            pltpu.sync_copy(m_hbm.at[pl.ds(mbase + j * _CH, _CH)], rows_v)
            pltpu.sync_copy(rows_v, acc_sh.at[idx_v], add=True)

        if tail:
            idxt_v, rowst_v = tail_refs
            offt = n_full * _CH
            pltpu.sync_copy(d_hbm.at[pl.ds(dbase + offt, tail)], idxt_v)
            pltpu.sync_copy(m_hbm.at[pl.ds(mbase + offt, tail)], rowst_v)
            pltpu.sync_copy(rowst_v, acc_sh.at[idxt_v], add=True)

        plsc.subcore_barrier()
        pltpu.sync_copy(acc_sh.at[pl.ds(s * nz, nz)], o_hbm.at[c, pl.ds(s * nz, nz)])

    return k(m, dst)


def _mlp_body(ea_ref, xj_ref, w1_ref, b1_ref, w2_ref, b2_ref, m_ref):
    te, nd = xj_ref.shape
    od = m_ref.shape[-1]
    h1p = jnp.dot(ea_ref[...], w1_ref[...], preferred_element_type=jnp.float32)
    h1 = h1p.reshape(te, b1_ref.shape[-1])
    h1 = _leaky(h1 + b1_ref[...])
    h2 = jnp.dot(h1.astype(w2_ref.dtype), w2_ref[...],
                 preferred_element_type=jnp.float32)
    h2 = _leaky(h2 + b2_ref[...])
    xj = xj_ref[...]
    cols = [
        jnp.sum(h2[:, o * nd:(o + 1) * nd] * xj, axis=-1, keepdims=True)
        for o in range(od)
    ]
    m_ref[...] = jnp.concatenate(cols, axis=-1)


def _tc_mlp(ea_w, x_j, w1big, b1f, w2p, b2p, od, e0):
    ec, nd = x_j.shape
    hd = b1f.shape[-1]
    ed8 = ea_w.shape[-1]
    assert ec % _TE == 0 and e0 % _TE == 0
    t0 = e0 // _TE
    return pl.pallas_call(
        _mlp_body,
        grid=(ec // _TE,),
        in_specs=[
            pl.BlockSpec((_TE // 8, ed8), lambda i: (i + t0, 0)),
            pl.BlockSpec((_TE, nd), lambda i: (i, 0)),
            pl.BlockSpec((ed8, 8 * hd), lambda i: (0, 0)),
            pl.BlockSpec((1, hd), lambda i: (0, 0)),
            pl.BlockSpec((nd, hd * od), lambda i: (0, 0)),
            pl.BlockSpec((1, hd * od), lambda i: (0, 0)),
        ],
        out_specs=pl.BlockSpec((_TE, od), lambda i: (i, 0)),
        out_shape=jax.ShapeDtypeStruct((ec, od), jnp.float32),
    )(ea_w, x_j, w1big, b1f, w2p, b2p)


def _finish_body(p0_ref, p1_ref, x_ref, rw_ref, b_ref, o_ref):
    root = jnp.dot(x_ref[...], rw_ref[...], preferred_element_type=jnp.float32)
    acc = p0_ref[0] + p0_ref[1] + p1_ref[0] + p1_ref[1]
    o_ref[...] = acc + root + b_ref[...]


def _tc_finish(partials0, partials1, x, root_w, bias):
    n, nd = x.shape
    od = root_w.shape[1]
    return pl.pallas_call(
        _finish_body,
        out_shape=jax.ShapeDtypeStruct((n, od), jnp.float32),
    )(partials0, partials1, x, root_w, bias.reshape(1, od))


def kernel(x, edge_index, edge_attr, batch, lin1_w, lin1_b, bn1_g, bn1_b,
           bn1_rm, bn1_rv, lin2_w, lin2_b, bn2_g, bn2_b, bn2_rm, bn2_rv,
           root_w, bias):
    n, nd = x.shape
    e, ed = edge_attr.shape
    hd = lin1_w.shape[1]
    od = root_w.shape[1]
    src2 = edge_index[0].reshape(1, e)
    dst = edge_index[1]

    s1 = bn1_g * lax.rsqrt(bn1_rv + _EPS)
    w1f = lin1_w * s1
    b1f = (s1 * (lin1_b - bn1_rm) + bn1_b).reshape(1, hd)
    s2 = bn2_g * lax.rsqrt(bn2_rv + _EPS)
    w2f = lin2_w * s2
    b2f = s2 * (lin2_b - bn2_rm) + bn2_b
    w2p = w2f.reshape(hd, hd, od).transpose(0, 2, 1).reshape(hd, hd * od)
    w2p = w2p.astype(jnp.bfloat16)
    b2p = b2f.reshape(hd, od).T.reshape(1, hd * od)

    w1big = jnp.zeros((8, ed, 8, hd), jnp.float32)
    for j in range(8):
        w1big = w1big.at[j, :, j, :].set(w1f)
    w1big = w1big.reshape(8 * ed, 8 * hd)
    ea_w = edge_attr.reshape(e // 8, 8 * ed)

    bounds = [(0, _SPLIT), (_SPLIT, e - _SPLIT)]
    partials = []
    x_js = [_sc_gather(x, src2, e0, ec) for e0, ec in bounds]
    for (e0, ec), x_j in zip(bounds, x_js):
        m = _tc_mlp(ea_w, x_j, w1big, b1f, w2p, b2p, od, e0)
        partials.append(_sc_scatter_add(m, dst, n, e0)[:, :n, :])
    return _tc_finish(partials[0], partials[1], x, root_w, bias)


# speedup vs baseline: 3.2466x; 1.0091x over previous
import functools

import jax
import jax.numpy as jnp
from jax import lax
from jax.experimental import pallas as pl
from jax.experimental.pallas import tpu as pltpu
from jax.experimental.pallas import tpu_sc as plsc

_EPS = 1e-5
_TE = 1280
_GW = 128
_CH = 128
_SPLIT = 81920


def _leaky(h):
    return jnp.maximum(h, 0.01 * h)


def _sc_gather(x, idx2, e0, ec):
    n, nd = x.shape
    assert ec % _GW == 0 and e0 % _GW == 0
    w0 = e0 // _GW
    mesh = plsc.VectorSubcoreMesh(core_axis_name="core", subcore_axis_name="subcore")

    @functools.partial(
        pl.kernel,
        out_type=jax.ShapeDtypeStruct((ec, nd), jnp.float32),
        mesh=mesh,
    )
    def k(x_hbm, i_hbm, o_hbm):
        def body(i_vmem, o_vmem):
            pltpu.sync_copy(x_hbm.at[i_vmem.at[0]], o_vmem)

        pltpu.emit_pipeline(
            body,
            grid=(ec // _GW,),
            in_specs=[pl.BlockSpec((1, _GW), index_map=lambda i: (0, i + w0))],
            out_specs=[pl.BlockSpec((_GW, nd), index_map=lambda i: (i, 0))],
            core_axis_name=("core", "subcore"),
            dimension_semantics=(pltpu.PARALLEL,),
        )(i_hbm, o_hbm)

    return k(x, idx2)


def _sc_scatter_add(m_wide, dst, n, e0, od):
    ec = m_wide.shape[0]
    e_half = ec // 2
    epc = e_half // 16
    n_full = epc // _CH
    tail = epc - n_full * _CH
    nz = -(-n // (16 * 8)) * 8
    npad = nz * 16
    assert epc % 8 == 0 and tail % 8 == 0 and e0 % 8 == 0
    mesh = plsc.VectorSubcoreMesh(core_axis_name="core", subcore_axis_name="subcore")

    scratch = [
        pltpu.VMEM((_CH,), jnp.int32),
        pltpu.VMEM((_CH, od), jnp.float32),
        pltpu.VMEM((nz, od), jnp.float32),
        pltpu.VMEM_SHARED((npad, od), jnp.float32),
    ]
    if tail:
        scratch += [
            pltpu.VMEM((tail,), jnp.int32),
            pltpu.VMEM((tail, od), jnp.float32),
        ]

    @functools.partial(
        pl.kernel,
        out_type=jax.ShapeDtypeStruct((2, npad, od), jnp.float32),
        mesh=mesh,
        scratch_types=scratch,
        compiler_params=pltpu.CompilerParams(use_tc_tiling_on_sc=False),
    )
    def k(m_hbm, d_hbm, o_hbm, idx_v, rows_v, z_v, acc_sh, *tail_refs):
        c = lax.axis_index("core")
        s = lax.axis_index("subcore")
        mbase = c * e_half + s * epc
        dbase = e0 + mbase

        @pl.loop(0, nz)
        def _(i):
            z_v[i, :] = jnp.zeros((od,), jnp.float32)

        pltpu.sync_copy(z_v, acc_sh.at[pl.ds(s * nz, nz)])
        plsc.subcore_barrier()

        @pl.loop(0, n_full)
        def _(j):
            pltpu.sync_copy(d_hbm.at[pl.ds(dbase + j * _CH, _CH)], idx_v)
            pltpu.sync_copy(m_hbm.at[pl.ds(mbase + j * _CH, _CH), pl.ds(0, od)],
                            rows_v)
            pltpu.sync_copy(rows_v, acc_sh.at[idx_v], add=True)

        if tail:
            idxt_v, rowst_v = tail_refs
            offt = n_full * _CH
            pltpu.sync_copy(d_hbm.at[pl.ds(dbase + offt, tail)], idxt_v)
            pltpu.sync_copy(m_hbm.at[pl.ds(mbase + offt, tail), pl.ds(0, od)],
                            rowst_v)
            pltpu.sync_copy(rowst_v, acc_sh.at[idxt_v], add=True)

        plsc.subcore_barrier()
        pltpu.sync_copy(acc_sh.at[pl.ds(s * nz, nz)], o_hbm.at[c, pl.ds(s * nz, nz)])

    return k(m_wide, dst)


def _mlp_body(ea_ref, xj_ref, w1_ref, b1_ref, w2_ref, b2_ref, m_ref):
    te, nd = xj_ref.shape
    od = b2_ref.shape[-1] // b1_ref.shape[-1]
    h1p = jnp.dot(ea_ref[...], w1_ref[...], preferred_element_type=jnp.float32)
    h1 = h1p.reshape(te, b1_ref.shape[-1])
    h1 = _leaky(h1 + b1_ref[...])
    h2 = jnp.dot(h1.astype(w2_ref.dtype), w2_ref[...],
                 preferred_element_type=jnp.float32)
    h2 = _leaky(h2 + b2_ref[...])
    xj = xj_ref[...]
    cols = [
        jnp.sum(h2[:, o * nd:(o + 1) * nd] * xj, axis=-1, keepdims=True)
        for o in range(od)
    ]
    cols.append(jnp.zeros((te, m_ref.shape[-1] - od), jnp.float32))
    m_ref[...] = jnp.concatenate(cols, axis=-1)


def _tc_mlp(ea_w, x_j, w1big, b1f, w2p, b2p, od):
    ec, nd = x_j.shape
    hd = b1f.shape[-1]
    ed8 = ea_w.shape[-1]
    assert ec % _TE == 0
    return pl.pallas_call(
        _mlp_body,
        grid=(ec // _TE,),
        in_specs=[
            pl.BlockSpec((_TE // 8, ed8), lambda i: (i, 0)),
            pl.BlockSpec((_TE, nd), lambda i: (i, 0)),
            pl.BlockSpec((ed8, 8 * hd), lambda i: (0, 0)),
            pl.BlockSpec((1, hd), lambda i: (0, 0)),
            pl.BlockSpec((nd, hd * od), lambda i: (0, 0)),
            pl.BlockSpec((1, hd * od), lambda i: (0, 0)),
        ],
        out_specs=pl.BlockSpec((_TE, 128), lambda i: (i, 0)),
        out_shape=jax.ShapeDtypeStruct((ec, 128), jnp.float32),
    )(ea_w, x_j, w1big, b1f, w2p, b2p)


def _make_finish_body(n):
    def _finish_body(p0_ref, p1_ref, x_ref, rw_ref, b_ref, o_ref):
        root = jnp.dot(x_ref[...], rw_ref[...], preferred_element_type=jnp.float32)
        acc = (p0_ref[0, :n, :] + p0_ref[1, :n, :]
               + p1_ref[0, :n, :] + p1_ref[1, :n, :])
        o_ref[...] = acc + root + b_ref[...]
    return _finish_body


def _tc_finish(partials0, partials1, x, root_w, bias):
    n, nd = x.shape
    od = root_w.shape[1]
    return pl.pallas_call(
        _make_finish_body(n),
        out_shape=jax.ShapeDtypeStruct((n, od), jnp.float32),
    )(partials0, partials1, x, root_w, bias.reshape(1, od))


def kernel(x, edge_index, edge_attr, batch, lin1_w, lin1_b, bn1_g, bn1_b,
           bn1_rm, bn1_rv, lin2_w, lin2_b, bn2_g, bn2_b, bn2_rm, bn2_rv,
           root_w, bias):
    n, nd = x.shape
    e, ed = edge_attr.shape
    hd = lin1_w.shape[1]
    od = root_w.shape[1]
    src2 = edge_index[0].reshape(1, e)
    dst = edge_index[1]

    s1 = bn1_g * lax.rsqrt(bn1_rv + _EPS)
    w1f = lin1_w * s1
    b1f = (s1 * (lin1_b - bn1_rm) + bn1_b).reshape(1, hd)
    s2 = bn2_g * lax.rsqrt(bn2_rv + _EPS)
    w2f = lin2_w * s2
    b2f = s2 * (lin2_b - bn2_rm) + bn2_b
    w2p = w2f.reshape(hd, hd, od).transpose(0, 2, 1).reshape(hd, hd * od)
    w2p = w2p.astype(jnp.bfloat16)
    b2p = b2f.reshape(hd, od).T.reshape(1, hd * od)

    w1big = jnp.zeros((8, ed, 8, hd), jnp.float32)
    for j in range(8):
        w1big = w1big.at[j, :, j, :].set(w1f)
    w1big = w1big.reshape(8 * ed, 8 * hd)

    bounds = [(0, _SPLIT), (_SPLIT, e - _SPLIT)]
    partials = []
    x_js = [_sc_gather(x, src2, e0, ec) for e0, ec in bounds]
    ea_ws = [edge_attr[e0:e0 + ec].reshape(ec // 8, 8 * ed) for e0, ec in bounds]
    for (e0, ec), x_j, ea_w in zip(bounds, x_js, ea_ws):
        m_wide = _tc_mlp(ea_w, x_j, w1big, b1f, w2p, b2p, od)
        partials.append(_sc_scatter_add(m_wide, dst, n, e0, od))
    return _tc_finish(partials[0], partials[1], x, root_w, bias)
